# Initial kernel scaffold; baseline (speedup 1.0000x reference)
import jax
import jax.numpy as jnp
from jax.experimental import pallas as pl


def kernel(x, edge_index, W1, a_s1, a_d1, b1, W2, a_s2, a_d2, b2):
    raise NotImplementedError("write your pallas kernel here")


# speedup vs baseline: 10.2570x; 10.2570x over previous
import functools

import jax
import jax.numpy as jnp
from jax import lax
from jax.experimental import pallas as pl
from jax.experimental.pallas import tpu as pltpu
from jax.experimental.pallas import tpu_sc as plsc

N = 10000
E = 320000
D = 128
DH = D // 2

N_PAD = 10240
EW = 20480
E_PAD = 16 * EW
C = 128
N_CHUNKS = EW // C
ROWS_PER_TILE = N_PAD // 16


_BLK = 1024


def _proj_body(x_ref, w_ref, as_ref, ad_ref, h_ref, pas_ref, pad_ref):
    h = jnp.dot(x_ref[...], w_ref[...], preferred_element_type=jnp.float32)
    h_ref[0] = h[:, :DH]
    h_ref[1] = h[:, DH:]
    pas_ref[...] = jnp.dot(h, as_ref[...], preferred_element_type=jnp.float32)
    pad_ref[...] = jnp.dot(h, ad_ref[...], preferred_element_type=jnp.float32)


def _proj(x, W, a_s, a_d):
    grid = (N_PAD // _BLK,)
    return pl.pallas_call(
        _proj_body,
        grid=grid,
        in_specs=[
            pl.BlockSpec((_BLK, D), lambda i: (i, 0)),
            pl.BlockSpec((D, D), lambda i: (0, 0)),
            pl.BlockSpec((D, 1), lambda i: (0, 0)),
            pl.BlockSpec((D, 1), lambda i: (0, 0)),
        ],
        out_specs=[
            pl.BlockSpec((2, _BLK, DH), lambda i: (0, i, 0)),
            pl.BlockSpec((_BLK, 1), lambda i: (i, 0)),
            pl.BlockSpec((_BLK, 1), lambda i: (i, 0)),
        ],
        out_shape=[
            jax.ShapeDtypeStruct((2, N_PAD, DH), jnp.float32),
            jax.ShapeDtypeStruct((N_PAD, 1), jnp.float32),
            jax.ShapeDtypeStruct((N_PAD, 1), jnp.float32),
        ],
    )(x, W, a_s, a_d)


def _combine_block(nref, dref, bref):
    g = jnp.concatenate([nref[0], nref[1]], axis=1)
    den = dref[:, 0:1]
    return jnp.where(den > 0.0, g / den, 0.0) + bref[...]


def _mid_body(n_ref, d_ref, b_ref, w_ref, as_ref, ad_ref,
              h_ref, pas_ref, pad_ref):
    o = _combine_block(n_ref, d_ref, b_ref)
    hin = jnp.maximum(o, 0.0)
    h = jnp.dot(hin, w_ref[...], preferred_element_type=jnp.float32)
    h_ref[0] = h[:, :DH]
    h_ref[1] = h[:, DH:]
    pas_ref[...] = jnp.dot(h, as_ref[...], preferred_element_type=jnp.float32)
    pad_ref[...] = jnp.dot(h, ad_ref[...], preferred_element_type=jnp.float32)


def _mid(num, den, b, W, a_s, a_d):
    grid = (N_PAD // _BLK,)
    return pl.pallas_call(
        _mid_body,
        grid=grid,
        in_specs=[
            pl.BlockSpec((2, _BLK, DH), lambda i: (0, i, 0)),
            pl.BlockSpec((_BLK, 16), lambda i: (i, 0)),
            pl.BlockSpec((1, D), lambda i: (0, 0)),
            pl.BlockSpec((D, D), lambda i: (0, 0)),
            pl.BlockSpec((D, 1), lambda i: (0, 0)),
            pl.BlockSpec((D, 1), lambda i: (0, 0)),
        ],
        out_specs=[
            pl.BlockSpec((2, _BLK, DH), lambda i: (0, i, 0)),
            pl.BlockSpec((_BLK, 1), lambda i: (i, 0)),
            pl.BlockSpec((_BLK, 1), lambda i: (i, 0)),
        ],
        out_shape=[
            jax.ShapeDtypeStruct((2, N_PAD, DH), jnp.float32),
            jax.ShapeDtypeStruct((N_PAD, 1), jnp.float32),
            jax.ShapeDtypeStruct((N_PAD, 1), jnp.float32),
        ],
    )(num, den, b, W, a_s, a_d)


def _final_body(n_ref, d_ref, b_ref, o_ref):
    o_ref[...] = _combine_block(n_ref, d_ref, b_ref)


def _final(num, den, b):
    grid = (N_PAD // _BLK,)
    return pl.pallas_call(
        _final_body,
        grid=grid,
        in_specs=[
            pl.BlockSpec((2, _BLK, DH), lambda i: (0, i, 0)),
            pl.BlockSpec((_BLK, 16), lambda i: (i, 0)),
            pl.BlockSpec((1, D), lambda i: (0, 0)),
        ],
        out_specs=pl.BlockSpec((_BLK, D), lambda i: (i, 0)),
        out_shape=jax.ShapeDtypeStruct((N_PAD, D), jnp.float32),
    )(num, den, b)


def _edge_body(h_hbm, as_hbm, ad_hbm, src_hbm, dst_hbm,
               num_out, den_out,
               as_v, ad_v, src_v, dst_v, ex_v, buf, den_buf,
               num_acc, den_acc, sem):
    cid = lax.axis_index("c")
    sid = lax.axis_index("s")
    base = sid * EW

    pltpu.sync_copy(as_hbm, as_v)
    pltpu.sync_copy(ad_hbm, ad_v)

    zeros16 = jnp.zeros((16,), jnp.float32)

    def zero_row(r, _):
        for j in range(DH // 16):
            buf[r, pl.ds(j * 16, 16)] = zeros16
        den_buf[r, pl.ds(0, 16)] = zeros16
        return 0

    lax.fori_loop(0, C, zero_row, 0)

    row0 = sid * ROWS_PER_TILE
    for k in range(ROWS_PER_TILE // C):
        pltpu.sync_copy(buf, num_acc.at[pl.ds(row0 + k * C, C)])
        pltpu.sync_copy(den_buf, den_acc.at[pl.ds(row0 + k * C, C)])
    plsc.subcore_barrier()

    col0 = jnp.zeros((16,), jnp.int32)
    lane = lax.iota(jnp.int32, 16)

    def chunk(c, _):
        cb = base + c * C
        pltpu.sync_copy(src_hbm.at[pl.ds(cb, C)], src_v)
        pltpu.sync_copy(dst_hbm.at[pl.ds(cb, C)], dst_v)
        pltpu.async_copy(h_hbm.at[cid].at[src_v], buf, sem).wait()

        def ex_step(j, _):
            s16 = src_v[pl.ds(j * 16, 16)]
            d16 = dst_v[pl.ds(j * 16, 16)]
            e = plsc.load_gather(as_v, [s16]) + plsc.load_gather(ad_v, [d16])
            e = jnp.where(e > 0.0, e, 0.2 * e)
            ex = jnp.exp(e)
            ex_v[pl.ds(j * 16, 16)] = ex
            plsc.store_scatter(den_buf, [j * 16 + lane, col0], ex)
            return 0

        lax.fori_loop(0, C // 16, ex_step, 0)

        def scale(j, _):
            ex16 = ex_v[pl.ds(j * 16, 16)]
            for l in range(16):
                exs = ex16[l]
                eb = j * 16 + l
                for r in range(DH // 16):
                    buf[eb, pl.ds(r * 16, 16)] = (
                        buf[eb, pl.ds(r * 16, 16)] * exs)
            return 0

        lax.fori_loop(0, C // 16, scale, 0)

        pltpu.sync_copy(buf, num_acc.at[dst_v], add=True)
        pltpu.sync_copy(den_buf, den_acc.at[dst_v], add=True)
        return 0

    lax.fori_loop(0, N_CHUNKS, chunk, 0)

    plsc.subcore_barrier()
    pltpu.sync_copy(num_acc.at[pl.ds(row0, ROWS_PER_TILE)],
                    num_out.at[cid, pl.ds(row0, ROWS_PER_TILE)])
    pltpu.sync_copy(den_acc.at[pl.ds(row0, ROWS_PER_TILE)],
                    den_out.at[cid, pl.ds(row0, ROWS_PER_TILE)])


@functools.partial(
    pl.kernel,
    out_type=[
        jax.ShapeDtypeStruct((2, N_PAD, DH), jnp.float32),
        jax.ShapeDtypeStruct((2, N_PAD, 16), jnp.float32),
    ],
    mesh=plsc.VectorSubcoreMesh(core_axis_name="c", subcore_axis_name="s",
                                num_cores=2, num_subcores=16),
    compiler_params=pltpu.CompilerParams(needs_layout_passes=False,
                                         use_tc_tiling_on_sc=False),
    scratch_types=[
        pltpu.VMEM((N_PAD,), jnp.float32),
        pltpu.VMEM((N_PAD,), jnp.float32),
        pltpu.VMEM((C,), jnp.int32),
        pltpu.VMEM((C,), jnp.int32),
        pltpu.VMEM((C,), jnp.float32),
        pltpu.VMEM((C, DH), jnp.float32),
        pltpu.VMEM((C, 16), jnp.float32),
        pltpu.VMEM_SHARED((N_PAD, DH), jnp.float32),
        pltpu.VMEM_SHARED((N_PAD, 16), jnp.float32),
        pltpu.SemaphoreType.DMA,
    ],
)
def _edge_pass(h, alpha_s, alpha_d, src, dst,
               num_out, den_out,
               as_v, ad_v, src_v, dst_v, ex_v, buf, den_buf,
               num_acc, den_acc, sem):
    _edge_body(h, alpha_s, alpha_d, src, dst, num_out, den_out,
               as_v, ad_v, src_v, dst_v, ex_v, buf, den_buf,
               num_acc, den_acc, sem)


def kernel(x, edge_index, W1, a_s1, a_d1, b1, W2, a_s2, a_d2, b2):
    x_pad = jnp.zeros((N_PAD, D), jnp.float32).at[:N].set(x)
    src = edge_index[0].astype(jnp.int32)
    dst = edge_index[1].astype(jnp.int32)
    pad = jnp.full((E_PAD - E,), N, jnp.int32)
    src = jnp.concatenate([src, pad])
    dst = jnp.concatenate([dst, pad])

    b1r = b1.reshape(1, D)
    b2r = b2.reshape(1, D)

    h1, pas1, pad1 = _proj(x_pad, W1, a_s1.reshape(D, 1), a_d1.reshape(D, 1))
    num1, den1 = _edge_pass(h1, pas1.reshape(N_PAD), pad1.reshape(N_PAD),
                            src, dst)
    h2, pas2, pad2 = _mid(num1, den1[0], b1r,
                          W2, a_s2.reshape(D, 1), a_d2.reshape(D, 1))
    num2, den2 = _edge_pass(h2, pas2.reshape(N_PAD), pad2.reshape(N_PAD),
                            src, dst)
    out = _final(num2, den2[0], b2r)
    return out[:N]


# speedup vs baseline: 18.2487x; 1.7791x over previous
import functools

import jax
import jax.numpy as jnp
from jax import lax
from jax.experimental import pallas as pl
from jax.experimental.pallas import tpu as pltpu
from jax.experimental.pallas import tpu_sc as plsc

N = 10000
E = 320000
D = 128
DH = D // 2

N_PAD = 10240
C = 128
N_CHUNKS = 159
EW = N_CHUNKS * C
E_PAD = 16 * EW
E_IDX = E_PAD + 3 * C
ROWS_PER_TILE = N_PAD // 16


_BLK = 1024


def _proj_body(x_ref, w_ref, as_ref, ad_ref, h_ref, pas_ref, pad_ref):
    h = jnp.dot(x_ref[...], w_ref[...], preferred_element_type=jnp.float32)
    h_ref[0] = h[:, :DH]
    h_ref[1] = h[:, DH:]
    pas_ref[...] = jnp.dot(h, as_ref[...], preferred_element_type=jnp.float32)
    pad_ref[...] = jnp.dot(h, ad_ref[...], preferred_element_type=jnp.float32)


def _proj(x, W, a_s, a_d):
    grid = (N_PAD // _BLK,)
    return pl.pallas_call(
        _proj_body,
        grid=grid,
        in_specs=[
            pl.BlockSpec((_BLK, D), lambda i: (i, 0)),
            pl.BlockSpec((D, D), lambda i: (0, 0)),
            pl.BlockSpec((D, 1), lambda i: (0, 0)),
            pl.BlockSpec((D, 1), lambda i: (0, 0)),
        ],
        out_specs=[
            pl.BlockSpec((2, _BLK, DH), lambda i: (0, i, 0)),
            pl.BlockSpec((_BLK, 1), lambda i: (i, 0)),
            pl.BlockSpec((_BLK, 1), lambda i: (i, 0)),
        ],
        out_shape=[
            jax.ShapeDtypeStruct((2, N_PAD, DH), jnp.float32),
            jax.ShapeDtypeStruct((N_PAD, 1), jnp.float32),
            jax.ShapeDtypeStruct((N_PAD, 1), jnp.float32),
        ],
    )(x, W, a_s, a_d)


def _combine_block(nref, dref, bref):
    g = jnp.concatenate([nref[0], nref[1]], axis=1)
    den = dref[:, 0:1]
    return jnp.where(den > 0.0, g / den, 0.0) + bref[...]


def _mid_body(n_ref, d_ref, b_ref, w_ref, as_ref, ad_ref,
              h_ref, pas_ref, pad_ref):
    o = _combine_block(n_ref, d_ref, b_ref)
    hin = jnp.maximum(o, 0.0)
    h = jnp.dot(hin, w_ref[...], preferred_element_type=jnp.float32)
    h_ref[0] = h[:, :DH]
    h_ref[1] = h[:, DH:]
    pas_ref[...] = jnp.dot(h, as_ref[...], preferred_element_type=jnp.float32)
    pad_ref[...] = jnp.dot(h, ad_ref[...], preferred_element_type=jnp.float32)


def _mid(num, den, b, W, a_s, a_d):
    grid = (N_PAD // _BLK,)
    return pl.pallas_call(
        _mid_body,
        grid=grid,
        in_specs=[
            pl.BlockSpec((2, _BLK, DH), lambda i: (0, i, 0)),
            pl.BlockSpec((_BLK, 16), lambda i: (i, 0)),
            pl.BlockSpec((1, D), lambda i: (0, 0)),
            pl.BlockSpec((D, D), lambda i: (0, 0)),
            pl.BlockSpec((D, 1), lambda i: (0, 0)),
            pl.BlockSpec((D, 1), lambda i: (0, 0)),
        ],
        out_specs=[
            pl.BlockSpec((2, _BLK, DH), lambda i: (0, i, 0)),
            pl.BlockSpec((_BLK, 1), lambda i: (i, 0)),
            pl.BlockSpec((_BLK, 1), lambda i: (i, 0)),
        ],
        out_shape=[
            jax.ShapeDtypeStruct((2, N_PAD, DH), jnp.float32),
            jax.ShapeDtypeStruct((N_PAD, 1), jnp.float32),
            jax.ShapeDtypeStruct((N_PAD, 1), jnp.float32),
        ],
    )(num, den, b, W, a_s, a_d)


def _final_body(n_ref, d_ref, b_ref, o_ref):
    o_ref[...] = _combine_block(n_ref, d_ref, b_ref)


def _final(num, den, b):
    grid = (N_PAD // _BLK,)
    return pl.pallas_call(
        _final_body,
        grid=grid,
        in_specs=[
            pl.BlockSpec((2, _BLK, DH), lambda i: (0, i, 0)),
            pl.BlockSpec((_BLK, 16), lambda i: (i, 0)),
            pl.BlockSpec((1, D), lambda i: (0, 0)),
        ],
        out_specs=pl.BlockSpec((_BLK, D), lambda i: (i, 0)),
        out_shape=jax.ShapeDtypeStruct((N_PAD, D), jnp.float32),
    )(num, den, b)


def _edge_body(h_hbm, as_hbm, ad_hbm, src_hbm, dst_hbm,
               num_out, den_out,
               as_v, ad_v, src_v, dst_v, sdst_v, ex_v, buf, den_buf,
               num_acc, den_acc,
               sem_i0, sem_i1, sem_g0, sem_g1,
               sem_sn0, sem_sn1, sem_sd0, sem_sd1):
    sem_i = (sem_i0, sem_i1)
    sem_g = (sem_g0, sem_g1)
    sem_sn = (sem_sn0, sem_sn1)
    sem_sd = (sem_sd0, sem_sd1)
    cid = lax.axis_index("c")
    sid = lax.axis_index("s")
    base = sid * EW

    pltpu.sync_copy(as_hbm, as_v)
    pltpu.sync_copy(ad_hbm, ad_v)

    zeros16 = jnp.zeros((16,), jnp.float32)

    def zero_row(r, _):
        for j in range(DH // 16):
            buf[0, r, pl.ds(j * 16, 16)] = zeros16
        for s in range(2):
            den_buf[s, r, pl.ds(0, 16)] = zeros16
        return 0

    lax.fori_loop(0, C, zero_row, 0)

    row0 = sid * ROWS_PER_TILE
    for k in range(ROWS_PER_TILE // C):
        pltpu.sync_copy(buf.at[0], num_acc.at[pl.ds(row0 + k * C, C)])
        pltpu.sync_copy(den_buf.at[0], den_acc.at[pl.ds(row0 + k * C, C)])
    plsc.subcore_barrier()

    col0 = jnp.zeros((16,), jnp.int32)
    lane = lax.iota(jnp.int32, 16)


    def issue_idx(g, s):
        cb = base + g * C
        pltpu.async_copy(src_hbm.at[pl.ds(cb, C)], src_v.at[s], sem_i[s])
        pltpu.async_copy(dst_hbm.at[pl.ds(cb, C)], dst_v.at[s], sem_i[s])

    def wait_idx(s):
        pltpu.make_async_copy(src_hbm.at[pl.ds(0, C)], src_v.at[s],
                              sem_i[s]).wait()
        pltpu.make_async_copy(dst_hbm.at[pl.ds(0, C)], dst_v.at[s],
                              sem_i[s]).wait()

    def issue_gather(s):
        pltpu.async_copy(h_hbm.at[cid].at[src_v.at[s]], buf.at[s], sem_g[s])

    def wait_gather(s):
        pltpu.make_async_copy(h_hbm.at[cid].at[src_v.at[s]], buf.at[s],
                              sem_g[s]).wait()

    def issue_scatter(s):
        pltpu.async_copy(buf.at[s], num_acc.at[sdst_v.at[s]], sem_sn[s],
                         add=True)
        pltpu.async_copy(den_buf.at[s], den_acc.at[sdst_v.at[s]], sem_sd[s],
                         add=True)

    def wait_scatter(s):
        pltpu.make_async_copy(buf.at[s], num_acc.at[sdst_v.at[s]],
                              sem_sn[s]).wait()
        pltpu.make_async_copy(den_buf.at[s], den_acc.at[sdst_v.at[s]],
                              sem_sd[s]).wait()

    def compute_ex(s):
        def ex_step(j, _):
            s16 = src_v[s, pl.ds(j * 16, 16)]
            d16 = dst_v[s, pl.ds(j * 16, 16)]
            e = plsc.load_gather(as_v, [s16]) + plsc.load_gather(ad_v, [d16])
            e = jnp.where(e > 0.0, e, 0.2 * e)
            ex = jnp.exp(e)
            ex_v[pl.ds(j * 16, 16)] = ex
            plsc.store_scatter(den_buf.at[s], [j * 16 + lane, col0], ex)
            sdst_v[s, pl.ds(j * 16, 16)] = d16
            return 0

        lax.fori_loop(0, C // 16, ex_step, 0)

    def scale(s):
        def sc_step(j, _):
            ex16 = ex_v[pl.ds(j * 16, 16)]
            for l in range(16):
                exs = ex16[l]
                eb = j * 16 + l
                for r in range(DH // 16):
                    buf[s, eb, pl.ds(r * 16, 16)] = (
                        buf[s, eb, pl.ds(r * 16, 16)] * exs)
            return 0

        lax.fori_loop(0, C // 16, sc_step, 0)

    issue_idx(0, 0)
    issue_idx(1, 1)
    wait_idx(0)
    issue_gather(0)

    def steady_chunk(g, p):
        pn = 1 - p
        wait_idx(pn)
        compute_ex(p)
        wait_gather(p)
        issue_idx(g + 2, p)
        wait_scatter(pn)
        issue_gather(pn)
        scale(p)
        issue_scatter(p)

    wait_idx(1)
    compute_ex(0)
    wait_gather(0)
    issue_idx(2, 0)
    issue_gather(1)
    scale(0)
    issue_scatter(0)
    wait_idx(0)
    compute_ex(1)
    wait_gather(1)
    issue_idx(3, 1)
    wait_scatter(0)
    issue_gather(0)
    scale(1)
    issue_scatter(1)

    def steady(i, _):
        for b in range(2):
            steady_chunk(2 + 2 * i + b, b)
        return 0

    lax.fori_loop(0, (N_CHUNKS - 3) // 2, steady, 0)

    compute_ex(0)
    wait_gather(0)
    wait_scatter(1)
    scale(0)
    issue_scatter(0)
    wait_scatter(0)

    plsc.subcore_barrier()
    pltpu.sync_copy(num_acc.at[pl.ds(row0, ROWS_PER_TILE)],
                    num_out.at[cid, pl.ds(row0, ROWS_PER_TILE)])
    pltpu.sync_copy(den_acc.at[pl.ds(row0, ROWS_PER_TILE)],
                    den_out.at[cid, pl.ds(row0, ROWS_PER_TILE)])


@functools.partial(
    pl.kernel,
    out_type=[
        jax.ShapeDtypeStruct((2, N_PAD, DH), jnp.float32),
        jax.ShapeDtypeStruct((2, N_PAD, 16), jnp.float32),
    ],
    mesh=plsc.VectorSubcoreMesh(core_axis_name="c", subcore_axis_name="s",
                                num_cores=2, num_subcores=16),
    compiler_params=pltpu.CompilerParams(needs_layout_passes=False,
                                         use_tc_tiling_on_sc=False),
    scratch_types=[
        pltpu.VMEM((N_PAD,), jnp.float32),
        pltpu.VMEM((N_PAD,), jnp.float32),
        pltpu.VMEM((2, C), jnp.int32),
        pltpu.VMEM((2, C), jnp.int32),
        pltpu.VMEM((2, C), jnp.int32),
        pltpu.VMEM((C,), jnp.float32),
        pltpu.VMEM((2, C, DH), jnp.float32),
        pltpu.VMEM((2, C, 16), jnp.float32),
        pltpu.VMEM_SHARED((N_PAD, DH), jnp.float32),
        pltpu.VMEM_SHARED((N_PAD, 16), jnp.float32),
        pltpu.SemaphoreType.DMA,
        pltpu.SemaphoreType.DMA,
        pltpu.SemaphoreType.DMA,
        pltpu.SemaphoreType.DMA,
        pltpu.SemaphoreType.DMA,
        pltpu.SemaphoreType.DMA,
        pltpu.SemaphoreType.DMA,
        pltpu.SemaphoreType.DMA,
    ],
)
def _edge_pass(h, alpha_s, alpha_d, src, dst,
               num_out, den_out,
               as_v, ad_v, src_v, dst_v, sdst_v, ex_v, buf, den_buf,
               num_acc, den_acc,
               sem_i0, sem_i1, sem_g0, sem_g1,
               sem_sn0, sem_sn1, sem_sd0, sem_sd1):
    _edge_body(h, alpha_s, alpha_d, src, dst, num_out, den_out,
               as_v, ad_v, src_v, dst_v, sdst_v, ex_v, buf, den_buf,
               num_acc, den_acc,
               sem_i0, sem_i1, sem_g0, sem_g1,
               sem_sn0, sem_sn1, sem_sd0, sem_sd1)


def kernel(x, edge_index, W1, a_s1, a_d1, b1, W2, a_s2, a_d2, b2):
    x_pad = jnp.zeros((N_PAD, D), jnp.float32).at[:N].set(x)
    src = edge_index[0].astype(jnp.int32)
    dst = edge_index[1].astype(jnp.int32)
    pad = jnp.full((E_IDX - E,), N, jnp.int32)
    src = jnp.concatenate([src, pad])
    dst = jnp.concatenate([dst, pad])

    b1r = b1.reshape(1, D)
    b2r = b2.reshape(1, D)

    h1, pas1, pad1 = _proj(x_pad, W1, a_s1.reshape(D, 1), a_d1.reshape(D, 1))
    num1, den1 = _edge_pass(h1, pas1.reshape(N_PAD), pad1.reshape(N_PAD),
                            src, dst)
    h2, pas2, pad2 = _mid(num1, den1[0], b1r,
                          W2, a_s2.reshape(D, 1), a_d2.reshape(D, 1))
    num2, den2 = _edge_pass(h2, pas2.reshape(N_PAD), pad2.reshape(N_PAD),
                            src, dst)
    out = _final(num2, den2[0], b2r)
    return out[:N]


# speedup vs baseline: 24.1055x; 1.3209x over previous
import functools

import jax
import jax.numpy as jnp
from jax import lax
from jax.experimental import pallas as pl
from jax.experimental.pallas import tpu as pltpu
from jax.experimental.pallas import tpu_sc as plsc

N = 10000
E = 320000
D = 128
DH = D // 2

N_PAD = 10240
C = 256
N_CHUNKS = 79
EW = N_CHUNKS * C
E_PAD = 16 * EW
E_IDX = E_PAD + 3 * C
ROWS_PER_TILE = N_PAD // 16


_BLK = 1024


def _proj_body(x_ref, w_ref, as_ref, ad_ref, h_ref, pas_ref, pad_ref):
    h = jnp.dot(x_ref[...], w_ref[...], preferred_element_type=jnp.float32)
    h_ref[0] = h[:, :DH]
    h_ref[1] = h[:, DH:]
    pas_ref[...] = jnp.dot(h, as_ref[...], preferred_element_type=jnp.float32)
    pad_ref[...] = jnp.dot(h, ad_ref[...], preferred_element_type=jnp.float32)


def _proj(x, W, a_s, a_d):
    grid = (N_PAD // _BLK,)
    return pl.pallas_call(
        _proj_body,
        grid=grid,
        in_specs=[
            pl.BlockSpec((_BLK, D), lambda i: (i, 0)),
            pl.BlockSpec((D, D), lambda i: (0, 0)),
            pl.BlockSpec((D, 1), lambda i: (0, 0)),
            pl.BlockSpec((D, 1), lambda i: (0, 0)),
        ],
        out_specs=[
            pl.BlockSpec((2, _BLK, DH), lambda i: (0, i, 0)),
            pl.BlockSpec((_BLK, 1), lambda i: (i, 0)),
            pl.BlockSpec((_BLK, 1), lambda i: (i, 0)),
        ],
        out_shape=[
            jax.ShapeDtypeStruct((2, N_PAD, DH), jnp.float32),
            jax.ShapeDtypeStruct((N_PAD, 1), jnp.float32),
            jax.ShapeDtypeStruct((N_PAD, 1), jnp.float32),
        ],
    )(x, W, a_s, a_d)


def _combine_block(nref, dref, bref):
    g = jnp.concatenate([nref[0], nref[1]], axis=1)
    den = dref[:, 0:1]
    return jnp.where(den > 0.0, g / den, 0.0) + bref[...]


def _mid_body(n_ref, d_ref, b_ref, w_ref, as_ref, ad_ref,
              h_ref, pas_ref, pad_ref):
    o = _combine_block(n_ref, d_ref, b_ref)
    hin = jnp.maximum(o, 0.0)
    h = jnp.dot(hin, w_ref[...], preferred_element_type=jnp.float32)
    h_ref[0] = h[:, :DH]
    h_ref[1] = h[:, DH:]
    pas_ref[...] = jnp.dot(h, as_ref[...], preferred_element_type=jnp.float32)
    pad_ref[...] = jnp.dot(h, ad_ref[...], preferred_element_type=jnp.float32)


def _mid(num, den, b, W, a_s, a_d):
    grid = (N_PAD // _BLK,)
    return pl.pallas_call(
        _mid_body,
        grid=grid,
        in_specs=[
            pl.BlockSpec((2, _BLK, DH), lambda i: (0, i, 0)),
            pl.BlockSpec((_BLK, 16), lambda i: (i, 0)),
            pl.BlockSpec((1, D), lambda i: (0, 0)),
            pl.BlockSpec((D, D), lambda i: (0, 0)),
            pl.BlockSpec((D, 1), lambda i: (0, 0)),
            pl.BlockSpec((D, 1), lambda i: (0, 0)),
        ],
        out_specs=[
            pl.BlockSpec((2, _BLK, DH), lambda i: (0, i, 0)),
            pl.BlockSpec((_BLK, 1), lambda i: (i, 0)),
            pl.BlockSpec((_BLK, 1), lambda i: (i, 0)),
        ],
        out_shape=[
            jax.ShapeDtypeStruct((2, N_PAD, DH), jnp.float32),
            jax.ShapeDtypeStruct((N_PAD, 1), jnp.float32),
            jax.ShapeDtypeStruct((N_PAD, 1), jnp.float32),
        ],
    )(num, den, b, W, a_s, a_d)


def _final_body(n_ref, d_ref, b_ref, o_ref):
    o_ref[...] = _combine_block(n_ref, d_ref, b_ref)


def _final(num, den, b):
    grid = (N_PAD // _BLK,)
    return pl.pallas_call(
        _final_body,
        grid=grid,
        in_specs=[
            pl.BlockSpec((2, _BLK, DH), lambda i: (0, i, 0)),
            pl.BlockSpec((_BLK, 16), lambda i: (i, 0)),
            pl.BlockSpec((1, D), lambda i: (0, 0)),
        ],
        out_specs=pl.BlockSpec((_BLK, D), lambda i: (i, 0)),
        out_shape=jax.ShapeDtypeStruct((N_PAD, D), jnp.float32),
    )(num, den, b)


def _edge_body(h_hbm, as_hbm, ad_hbm, src_hbm, dst_hbm,
               num_out, den_out,
               as_v, ad_v, src_v, dst_v, sdst_v, ex_v, buf, den_buf,
               num_acc, den_acc,
               sem_i0, sem_i1, sem_g0, sem_g1,
               sem_sn0, sem_sn1, sem_sd0, sem_sd1):
    sem_i = (sem_i0, sem_i1)
    sem_g = (sem_g0, sem_g1)
    sem_sn = (sem_sn0, sem_sn1)
    sem_sd = (sem_sd0, sem_sd1)
    cid = lax.axis_index("c")
    sid = lax.axis_index("s")
    base = sid * EW

    pltpu.sync_copy(as_hbm, as_v)
    pltpu.sync_copy(ad_hbm, ad_v)

    zeros16 = jnp.zeros((16,), jnp.float32)

    def zero_row(r, _):
        for j in range(DH // 16):
            buf[0, r, pl.ds(j * 16, 16)] = zeros16
        for s in range(2):
            den_buf[s, r, pl.ds(0, 16)] = zeros16
        return 0

    lax.fori_loop(0, C, zero_row, 0)

    row0 = sid * ROWS_PER_TILE
    for k in range(ROWS_PER_TILE // C):
        pltpu.sync_copy(buf.at[0], num_acc.at[pl.ds(row0 + k * C, C)])
        pltpu.sync_copy(den_buf.at[0], den_acc.at[pl.ds(row0 + k * C, C)])
    _rem = ROWS_PER_TILE % C
    if _rem:
        r0 = row0 + (ROWS_PER_TILE // C) * C
        pltpu.sync_copy(buf.at[0, pl.ds(0, _rem)],
                        num_acc.at[pl.ds(r0, _rem)])
        pltpu.sync_copy(den_buf.at[0, pl.ds(0, _rem)],
                        den_acc.at[pl.ds(r0, _rem)])
    plsc.subcore_barrier()

    col0 = jnp.zeros((16,), jnp.int32)
    lane = lax.iota(jnp.int32, 16)


    def issue_idx(g, s):
        cb = base + g * C
        pltpu.async_copy(src_hbm.at[pl.ds(cb, C)], src_v.at[s], sem_i[s])
        pltpu.async_copy(dst_hbm.at[pl.ds(cb, C)], dst_v.at[s], sem_i[s])

    def wait_idx(s):
        pltpu.make_async_copy(src_hbm.at[pl.ds(0, C)], src_v.at[s],
                              sem_i[s]).wait()
        pltpu.make_async_copy(dst_hbm.at[pl.ds(0, C)], dst_v.at[s],
                              sem_i[s]).wait()

    def issue_gather(s):
        pltpu.async_copy(h_hbm.at[cid].at[src_v.at[s]], buf.at[s], sem_g[s])

    def wait_gather(s):
        pltpu.make_async_copy(h_hbm.at[cid].at[src_v.at[s]], buf.at[s],
                              sem_g[s]).wait()

    def issue_scatter(s):
        pltpu.async_copy(buf.at[s], num_acc.at[sdst_v.at[s]], sem_sn[s],
                         add=True)
        pltpu.async_copy(den_buf.at[s], den_acc.at[sdst_v.at[s]], sem_sd[s],
                         add=True)

    def wait_scatter(s):
        pltpu.make_async_copy(buf.at[s], num_acc.at[sdst_v.at[s]],
                              sem_sn[s]).wait()
        pltpu.make_async_copy(den_buf.at[s], den_acc.at[sdst_v.at[s]],
                              sem_sd[s]).wait()

    def compute_ex(s):
        def ex_step(j, _):
            s16 = src_v[s, pl.ds(j * 16, 16)]
            d16 = dst_v[s, pl.ds(j * 16, 16)]
            e = plsc.load_gather(as_v, [s16]) + plsc.load_gather(ad_v, [d16])
            e = jnp.where(e > 0.0, e, 0.2 * e)
            ex = jnp.exp(e)
            ex_v[pl.ds(j * 16, 16)] = ex
            plsc.store_scatter(den_buf.at[s], [j * 16 + lane, col0], ex)
            sdst_v[s, pl.ds(j * 16, 16)] = d16
            return 0

        lax.fori_loop(0, C // 16, ex_step, 0)

    def scale(s):
        def sc_step(j, _):
            ex16 = ex_v[pl.ds(j * 16, 16)]
            for l in range(16):
                exs = ex16[l]
                eb = j * 16 + l
                for r in range(DH // 16):
                    buf[s, eb, pl.ds(r * 16, 16)] = (
                        buf[s, eb, pl.ds(r * 16, 16)] * exs)
            return 0

        lax.fori_loop(0, C // 16, sc_step, 0)

    issue_idx(0, 0)
    issue_idx(1, 1)
    wait_idx(0)
    issue_gather(0)

    def steady_chunk(g, p):
        pn = 1 - p
        wait_idx(pn)
        compute_ex(p)
        wait_gather(p)
        issue_idx(g + 2, p)
        wait_scatter(pn)
        issue_gather(pn)
        scale(p)
        issue_scatter(p)

    wait_idx(1)
    compute_ex(0)
    wait_gather(0)
    issue_idx(2, 0)
    issue_gather(1)
    scale(0)
    issue_scatter(0)
    wait_idx(0)
    compute_ex(1)
    wait_gather(1)
    issue_idx(3, 1)
    wait_scatter(0)
    issue_gather(0)
    scale(1)
    issue_scatter(1)

    def steady(i, _):
        for b in range(2):
            steady_chunk(2 + 2 * i + b, b)
        return 0

    lax.fori_loop(0, (N_CHUNKS - 3) // 2, steady, 0)

    compute_ex(0)
    wait_gather(0)
    wait_scatter(1)
    scale(0)
    issue_scatter(0)
    wait_scatter(0)

    plsc.subcore_barrier()
    pltpu.sync_copy(num_acc.at[pl.ds(row0, ROWS_PER_TILE)],
                    num_out.at[cid, pl.ds(row0, ROWS_PER_TILE)])
    pltpu.sync_copy(den_acc.at[pl.ds(row0, ROWS_PER_TILE)],
                    den_out.at[cid, pl.ds(row0, ROWS_PER_TILE)])


@functools.partial(
    pl.kernel,
    out_type=[
        jax.ShapeDtypeStruct((2, N_PAD, DH), jnp.float32),
        jax.ShapeDtypeStruct((2, N_PAD, 16), jnp.float32),
    ],
    mesh=plsc.VectorSubcoreMesh(core_axis_name="c", subcore_axis_name="s",
                                num_cores=2, num_subcores=16),
    compiler_params=pltpu.CompilerParams(needs_layout_passes=False,
                                         use_tc_tiling_on_sc=False),
    scratch_types=[
        pltpu.VMEM((N_PAD,), jnp.float32),
        pltpu.VMEM((N_PAD,), jnp.float32),
        pltpu.VMEM((2, C), jnp.int32),
        pltpu.VMEM((2, C), jnp.int32),
        pltpu.VMEM((2, C), jnp.int32),
        pltpu.VMEM((C,), jnp.float32),
        pltpu.VMEM((2, C, DH), jnp.float32),
        pltpu.VMEM((2, C, 16), jnp.float32),
        pltpu.VMEM_SHARED((N_PAD, DH), jnp.float32),
        pltpu.VMEM_SHARED((N_PAD, 16), jnp.float32),
        pltpu.SemaphoreType.DMA,
        pltpu.SemaphoreType.DMA,
        pltpu.SemaphoreType.DMA,
        pltpu.SemaphoreType.DMA,
        pltpu.SemaphoreType.DMA,
        pltpu.SemaphoreType.DMA,
        pltpu.SemaphoreType.DMA,
        pltpu.SemaphoreType.DMA,
    ],
)
def _edge_pass(h, alpha_s, alpha_d, src, dst,
               num_out, den_out,
               as_v, ad_v, src_v, dst_v, sdst_v, ex_v, buf, den_buf,
               num_acc, den_acc,
               sem_i0, sem_i1, sem_g0, sem_g1,
               sem_sn0, sem_sn1, sem_sd0, sem_sd1):
    _edge_body(h, alpha_s, alpha_d, src, dst, num_out, den_out,
               as_v, ad_v, src_v, dst_v, sdst_v, ex_v, buf, den_buf,
               num_acc, den_acc,
               sem_i0, sem_i1, sem_g0, sem_g1,
               sem_sn0, sem_sn1, sem_sd0, sem_sd1)


def kernel(x, edge_index, W1, a_s1, a_d1, b1, W2, a_s2, a_d2, b2):
    x_pad = jnp.zeros((N_PAD, D), jnp.float32).at[:N].set(x)
    src = edge_index[0].astype(jnp.int32)
    dst = edge_index[1].astype(jnp.int32)
    pad = jnp.full((E_IDX - E,), N, jnp.int32)
    src = jnp.concatenate([src, pad])
    dst = jnp.concatenate([dst, pad])

    b1r = b1.reshape(1, D)
    b2r = b2.reshape(1, D)

    h1, pas1, pad1 = _proj(x_pad, W1, a_s1.reshape(D, 1), a_d1.reshape(D, 1))
    num1, den1 = _edge_pass(h1, pas1.reshape(N_PAD), pad1.reshape(N_PAD),
                            src, dst)
    h2, pas2, pad2 = _mid(num1, den1[0], b1r,
                          W2, a_s2.reshape(D, 1), a_d2.reshape(D, 1))
    num2, den2 = _edge_pass(h2, pas2.reshape(N_PAD), pad2.reshape(N_PAD),
                            src, dst)
    out = _final(num2, den2[0], b2r)
    return out[:N]


# speedup vs baseline: 27.4596x; 1.1391x over previous
import functools

import jax
import jax.numpy as jnp
from jax import lax
from jax.experimental import pallas as pl
from jax.experimental.pallas import tpu as pltpu
from jax.experimental.pallas import tpu_sc as plsc

N = 10000
E = 320000
D = 128
DH = D // 2
DP = DH + 16

N_PAD = 10240
C = 256
N_CHUNKS = 79
EW = N_CHUNKS * C
E_PAD = 16 * EW
N_BLKS = E_PAD // C + 3
ROWS_PER_TILE = N_PAD // 16


_BLK = 1024


def _pad_halves(h):
    one = jnp.ones((_BLK, 1), jnp.float32)
    zer = jnp.zeros((_BLK, 15), jnp.float32)
    return (jnp.concatenate([h[:, :DH], one, zer], axis=1),
            jnp.concatenate([h[:, DH:], one, zer], axis=1))


def _proj_body(x_ref, w_ref, as_ref, ad_ref, h_ref, pas_ref, pad_ref):
    h = jnp.dot(x_ref[...], w_ref[...], preferred_element_type=jnp.float32)
    h_ref[0], h_ref[1] = _pad_halves(h)
    pas_ref[...] = jnp.dot(h, as_ref[...], preferred_element_type=jnp.float32)
    pad_ref[...] = jnp.dot(h, ad_ref[...], preferred_element_type=jnp.float32)


def _proj(x, W, a_s, a_d):
    grid = (N_PAD // _BLK,)
    return pl.pallas_call(
        _proj_body,
        grid=grid,
        in_specs=[
            pl.BlockSpec((_BLK, D), lambda i: (i, 0)),
            pl.BlockSpec((D, D), lambda i: (0, 0)),
            pl.BlockSpec((D, 1), lambda i: (0, 0)),
            pl.BlockSpec((D, 1), lambda i: (0, 0)),
        ],
        out_specs=[
            pl.BlockSpec((2, _BLK, DP), lambda i: (0, i, 0)),
            pl.BlockSpec((_BLK, 1), lambda i: (i, 0)),
            pl.BlockSpec((_BLK, 1), lambda i: (i, 0)),
        ],
        out_shape=[
            jax.ShapeDtypeStruct((2, N_PAD, DP), jnp.float32),
            jax.ShapeDtypeStruct((N_PAD, 1), jnp.float32),
            jax.ShapeDtypeStruct((N_PAD, 1), jnp.float32),
        ],
    )(x, W, a_s, a_d)


def _combine_block(nref, bref):
    g = jnp.concatenate([nref[0, :, :DH], nref[1, :, :DH]], axis=1)
    den = nref[0, :, DH:DH + 1]
    return jnp.where(den > 0.0, g / den, 0.0) + bref[...]


def _mid_body(n_ref, b_ref, w_ref, as_ref, ad_ref, h_ref, pas_ref, pad_ref):
    o = _combine_block(n_ref, b_ref)
    hin = jnp.maximum(o, 0.0)
    h = jnp.dot(hin, w_ref[...], preferred_element_type=jnp.float32)
    h_ref[0], h_ref[1] = _pad_halves(h)
    pas_ref[...] = jnp.dot(h, as_ref[...], preferred_element_type=jnp.float32)
    pad_ref[...] = jnp.dot(h, ad_ref[...], preferred_element_type=jnp.float32)


def _mid(num, b, W, a_s, a_d):
    grid = (N_PAD // _BLK,)
    return pl.pallas_call(
        _mid_body,
        grid=grid,
        in_specs=[
            pl.BlockSpec((2, _BLK, DP), lambda i: (0, i, 0)),
            pl.BlockSpec((1, D), lambda i: (0, 0)),
            pl.BlockSpec((D, D), lambda i: (0, 0)),
            pl.BlockSpec((D, 1), lambda i: (0, 0)),
            pl.BlockSpec((D, 1), lambda i: (0, 0)),
        ],
        out_specs=[
            pl.BlockSpec((2, _BLK, DP), lambda i: (0, i, 0)),
            pl.BlockSpec((_BLK, 1), lambda i: (i, 0)),
            pl.BlockSpec((_BLK, 1), lambda i: (i, 0)),
        ],
        out_shape=[
            jax.ShapeDtypeStruct((2, N_PAD, DP), jnp.float32),
            jax.ShapeDtypeStruct((N_PAD, 1), jnp.float32),
            jax.ShapeDtypeStruct((N_PAD, 1), jnp.float32),
        ],
    )(num, b, W, a_s, a_d)


def _final_body(n_ref, b_ref, o_ref):
    o_ref[...] = _combine_block(n_ref, b_ref)


def _final(num, b):
    grid = (N_PAD // _BLK,)
    return pl.pallas_call(
        _final_body,
        grid=grid,
        in_specs=[
            pl.BlockSpec((2, _BLK, DP), lambda i: (0, i, 0)),
            pl.BlockSpec((1, D), lambda i: (0, 0)),
        ],
        out_specs=pl.BlockSpec((_BLK, D), lambda i: (i, 0)),
        out_shape=jax.ShapeDtypeStruct((N_PAD, D), jnp.float32),
    )(num, b)


def _edge_body(h_hbm, as_hbm, ad_hbm, idx_hbm,
               num_out,
               as_v, ad_v, iv, sdst_v, ex_v, buf,
               num_acc,
               sem_i0, sem_i1, sem_g0, sem_g1, sem_s0, sem_s1):
    sem_i = (sem_i0, sem_i1)
    sem_g = (sem_g0, sem_g1)
    sem_s = (sem_s0, sem_s1)
    cid = lax.axis_index("c")
    sid = lax.axis_index("s")
    blk0 = sid * N_CHUNKS

    pltpu.sync_copy(as_hbm, as_v)
    pltpu.sync_copy(ad_hbm, ad_v)

    zeros16 = jnp.zeros((16,), jnp.float32)

    def zero_row(r, _):
        for j in range(DP // 16):
            buf[0, r, pl.ds(j * 16, 16)] = zeros16
        return 0

    lax.fori_loop(0, C, zero_row, 0)

    row0 = sid * ROWS_PER_TILE
    for k in range(ROWS_PER_TILE // C):
        pltpu.sync_copy(buf.at[0], num_acc.at[pl.ds(row0 + k * C, C)])
    _rem = ROWS_PER_TILE % C
    if _rem:
        r0 = row0 + (ROWS_PER_TILE // C) * C
        pltpu.sync_copy(buf.at[0, pl.ds(0, _rem)],
                        num_acc.at[pl.ds(r0, _rem)])
    plsc.subcore_barrier()


    def issue_idx(g, s):
        pltpu.async_copy(idx_hbm.at[blk0 + g], iv.at[s], sem_i[s])

    def wait_idx(s):
        pltpu.make_async_copy(idx_hbm.at[0], iv.at[s], sem_i[s]).wait()

    def issue_gather(s):
        pltpu.async_copy(h_hbm.at[cid].at[iv.at[s, 0]], buf.at[s], sem_g[s])

    def wait_gather(s):
        pltpu.make_async_copy(h_hbm.at[cid].at[iv.at[s, 0]], buf.at[s],
                              sem_g[s]).wait()

    def issue_scatter(s):
        pltpu.async_copy(buf.at[s], num_acc.at[sdst_v.at[s]], sem_s[s],
                         add=True)

    def wait_scatter(s):
        pltpu.make_async_copy(buf.at[s], num_acc.at[sdst_v.at[s]],
                              sem_s[s]).wait()

    def compute_ex(s):
        def ex_step(j, _):
            s16 = iv[s, 0, pl.ds(j * 16, 16)]
            d16 = iv[s, 1, pl.ds(j * 16, 16)]
            e = plsc.load_gather(as_v, [s16]) + plsc.load_gather(ad_v, [d16])
            e = jnp.where(e > 0.0, e, 0.2 * e)
            ex_v[pl.ds(j * 16, 16)] = jnp.exp(e)
            sdst_v[s, pl.ds(j * 16, 16)] = d16
            return 0

        lax.fori_loop(0, C // 16, ex_step, 0)

    def scale(s):
        def sc_step(j, _):
            ex16 = ex_v[pl.ds(j * 16, 16)]
            for l in range(16):
                exs = ex16[l]
                eb = j * 16 + l
                for r in range(DP // 16):
                    buf[s, eb, pl.ds(r * 16, 16)] = (
                        buf[s, eb, pl.ds(r * 16, 16)] * exs)
            return 0

        lax.fori_loop(0, C // 16, sc_step, 0)

    issue_idx(0, 0)
    issue_idx(1, 1)
    wait_idx(0)
    issue_gather(0)

    def steady_chunk(g, p):
        pn = 1 - p
        wait_idx(pn)
        compute_ex(p)
        wait_gather(p)
        issue_idx(g + 2, p)
        wait_scatter(pn)
        issue_gather(pn)
        scale(p)
        issue_scatter(p)

    wait_idx(1)
    compute_ex(0)
    wait_gather(0)
    issue_idx(2, 0)
    issue_gather(1)
    scale(0)
    issue_scatter(0)
    wait_idx(0)
    compute_ex(1)
    wait_gather(1)
    issue_idx(3, 1)
    wait_scatter(0)
    issue_gather(0)
    scale(1)
    issue_scatter(1)

    def steady(i, _):
        for b in range(2):
            steady_chunk(2 + 2 * i + b, b)
        return 0

    lax.fori_loop(0, (N_CHUNKS - 3) // 2, steady, 0)

    compute_ex(0)
    wait_gather(0)
    wait_scatter(1)
    scale(0)
    issue_scatter(0)
    wait_scatter(0)

    plsc.subcore_barrier()
    pltpu.sync_copy(num_acc.at[pl.ds(row0, ROWS_PER_TILE)],
                    num_out.at[cid, pl.ds(row0, ROWS_PER_TILE)])


@functools.partial(
    pl.kernel,
    out_type=[
        jax.ShapeDtypeStruct((2, N_PAD, DP), jnp.float32),
    ],
    mesh=plsc.VectorSubcoreMesh(core_axis_name="c", subcore_axis_name="s",
                                num_cores=2, num_subcores=16),
    compiler_params=pltpu.CompilerParams(needs_layout_passes=False,
                                         use_tc_tiling_on_sc=False),
    scratch_types=[
        pltpu.VMEM((N_PAD,), jnp.float32),
        pltpu.VMEM((N_PAD,), jnp.float32),
        pltpu.VMEM((2, 2, C), jnp.int32),
        pltpu.VMEM((2, C), jnp.int32),
        pltpu.VMEM((C,), jnp.float32),
        pltpu.VMEM((2, C, DP), jnp.float32),
        pltpu.VMEM_SHARED((N_PAD, DP), jnp.float32),
        pltpu.SemaphoreType.DMA,
        pltpu.SemaphoreType.DMA,
        pltpu.SemaphoreType.DMA,
        pltpu.SemaphoreType.DMA,
        pltpu.SemaphoreType.DMA,
        pltpu.SemaphoreType.DMA,
    ],
)
def _edge_pass(h, alpha_s, alpha_d, idx,
               num_out,
               as_v, ad_v, iv, sdst_v, ex_v, buf,
               num_acc,
               sem_i0, sem_i1, sem_g0, sem_g1, sem_s0, sem_s1):
    _edge_body(h, alpha_s, alpha_d, idx, num_out,
               as_v, ad_v, iv, sdst_v, ex_v, buf, num_acc,
               sem_i0, sem_i1, sem_g0, sem_g1, sem_s0, sem_s1)


def kernel(x, edge_index, W1, a_s1, a_d1, b1, W2, a_s2, a_d2, b2):
    x_pad = jnp.zeros((N_PAD, D), jnp.float32).at[:N].set(x)
    src = edge_index[0].astype(jnp.int32)
    dst = edge_index[1].astype(jnp.int32)
    pad = jnp.full((N_BLKS * C - E,), N, jnp.int32)
    src = jnp.concatenate([src, pad]).reshape(N_BLKS, 1, C)
    dst = jnp.concatenate([dst, pad]).reshape(N_BLKS, 1, C)
    idx = jnp.concatenate([src, dst], axis=1)

    b1r = b1.reshape(1, D)
    b2r = b2.reshape(1, D)

    h1, pas1, pad1 = _proj(x_pad, W1, a_s1.reshape(D, 1), a_d1.reshape(D, 1))
    num1, = _edge_pass(h1, pas1.reshape(N_PAD), pad1.reshape(N_PAD), idx)
    h2, pas2, pad2 = _mid(num1, b1r, W2, a_s2.reshape(D, 1), a_d2.reshape(D, 1))
    num2, = _edge_pass(h2, pas2.reshape(N_PAD), pad2.reshape(N_PAD), idx)
    out = _final(num2, b2r)
    return out[:N]


# speedup vs baseline: 30.2140x; 1.1003x over previous
import functools

import jax
import jax.numpy as jnp
from jax import lax
from jax.experimental import pallas as pl
from jax.experimental.pallas import tpu as pltpu
from jax.experimental.pallas import tpu_sc as plsc

N = 10000
E = 320000
D = 128
DH = D // 2
DP = DH + 16

N_PAD = 10240
C = 320
N_CHUNKS = 63
EW = N_CHUNKS * C
E_PAD = 16 * EW
N_BLKS = E_PAD // C + 3
ROWS_PER_TILE = N_PAD // 16


_BLK = 1024


def _pad_halves(h):
    one = jnp.ones((_BLK, 1), jnp.float32)
    zer = jnp.zeros((_BLK, 15), jnp.float32)
    return (jnp.concatenate([h[:, :DH], one, zer], axis=1),
            jnp.concatenate([h[:, DH:], one, zer], axis=1))


def _proj_body(x_ref, w_ref, as_ref, ad_ref, h_ref, pas_ref, pad_ref):
    h = jnp.dot(x_ref[...], w_ref[...], preferred_element_type=jnp.float32)
    h_ref[0], h_ref[1] = _pad_halves(h)
    pas_ref[...] = jnp.dot(h, as_ref[...], preferred_element_type=jnp.float32)
    pad_ref[...] = jnp.dot(h, ad_ref[...], preferred_element_type=jnp.float32)


def _proj(x, W, a_s, a_d):
    grid = (N_PAD // _BLK,)
    return pl.pallas_call(
        _proj_body,
        grid=grid,
        in_specs=[
            pl.BlockSpec((_BLK, D), lambda i: (i, 0)),
            pl.BlockSpec((D, D), lambda i: (0, 0)),
            pl.BlockSpec((D, 1), lambda i: (0, 0)),
            pl.BlockSpec((D, 1), lambda i: (0, 0)),
        ],
        out_specs=[
            pl.BlockSpec((2, _BLK, DP), lambda i: (0, i, 0)),
            pl.BlockSpec((_BLK, 1), lambda i: (i, 0)),
            pl.BlockSpec((_BLK, 1), lambda i: (i, 0)),
        ],
        out_shape=[
            jax.ShapeDtypeStruct((2, N_PAD, DP), jnp.float32),
            jax.ShapeDtypeStruct((N_PAD, 1), jnp.float32),
            jax.ShapeDtypeStruct((N_PAD, 1), jnp.float32),
        ],
    )(x, W, a_s, a_d)


def _combine_block(nref, bref):
    g = jnp.concatenate([nref[0, :, :DH], nref[1, :, :DH]], axis=1)
    den = nref[0, :, DH:DH + 1]
    return jnp.where(den > 0.0, g / den, 0.0) + bref[...]


def _mid_body(n_ref, b_ref, w_ref, as_ref, ad_ref, h_ref, pas_ref, pad_ref):
    o = _combine_block(n_ref, b_ref)
    hin = jnp.maximum(o, 0.0)
    h = jnp.dot(hin, w_ref[...], preferred_element_type=jnp.float32)
    h_ref[0], h_ref[1] = _pad_halves(h)
    pas_ref[...] = jnp.dot(h, as_ref[...], preferred_element_type=jnp.float32)
    pad_ref[...] = jnp.dot(h, ad_ref[...], preferred_element_type=jnp.float32)


def _mid(num, b, W, a_s, a_d):
    grid = (N_PAD // _BLK,)
    return pl.pallas_call(
        _mid_body,
        grid=grid,
        in_specs=[
            pl.BlockSpec((2, _BLK, DP), lambda i: (0, i, 0)),
            pl.BlockSpec((1, D), lambda i: (0, 0)),
            pl.BlockSpec((D, D), lambda i: (0, 0)),
            pl.BlockSpec((D, 1), lambda i: (0, 0)),
            pl.BlockSpec((D, 1), lambda i: (0, 0)),
        ],
        out_specs=[
            pl.BlockSpec((2, _BLK, DP), lambda i: (0, i, 0)),
            pl.BlockSpec((_BLK, 1), lambda i: (i, 0)),
            pl.BlockSpec((_BLK, 1), lambda i: (i, 0)),
        ],
        out_shape=[
            jax.ShapeDtypeStruct((2, N_PAD, DP), jnp.float32),
            jax.ShapeDtypeStruct((N_PAD, 1), jnp.float32),
            jax.ShapeDtypeStruct((N_PAD, 1), jnp.float32),
        ],
    )(num, b, W, a_s, a_d)


def _final_body(n_ref, b_ref, o_ref):
    o_ref[...] = _combine_block(n_ref, b_ref)


def _final(num, b):
    grid = (N_PAD // _BLK,)
    return pl.pallas_call(
        _final_body,
        grid=grid,
        in_specs=[
            pl.BlockSpec((2, _BLK, DP), lambda i: (0, i, 0)),
            pl.BlockSpec((1, D), lambda i: (0, 0)),
        ],
        out_specs=pl.BlockSpec((_BLK, D), lambda i: (i, 0)),
        out_shape=jax.ShapeDtypeStruct((N_PAD, D), jnp.float32),
    )(num, b)


def _edge_body(h_hbm, as_hbm, ad_hbm, idx_hbm,
               num_out,
               as_v, ad_v, iv, sdst_v, ex_v, buf,
               num_acc,
               sem_i0, sem_i1, sem_g0, sem_g1, sem_s0, sem_s1):
    sem_i = (sem_i0, sem_i1)
    sem_g = (sem_g0, sem_g1)
    sem_s = (sem_s0, sem_s1)
    cid = lax.axis_index("c")
    sid = lax.axis_index("s")
    blk0 = sid * N_CHUNKS

    pltpu.sync_copy(as_hbm, as_v)
    pltpu.sync_copy(ad_hbm, ad_v)

    zeros16 = jnp.zeros((16,), jnp.float32)

    def zero_row(r, _):
        for j in range(DP // 16):
            buf[0, r, pl.ds(j * 16, 16)] = zeros16
        return 0

    lax.fori_loop(0, C, zero_row, 0)

    row0 = sid * ROWS_PER_TILE
    for k in range(ROWS_PER_TILE // C):
        pltpu.sync_copy(buf.at[0], num_acc.at[pl.ds(row0 + k * C, C)])
    _rem = ROWS_PER_TILE % C
    if _rem:
        r0 = row0 + (ROWS_PER_TILE // C) * C
        pltpu.sync_copy(buf.at[0, pl.ds(0, _rem)],
                        num_acc.at[pl.ds(r0, _rem)])
    plsc.subcore_barrier()


    def issue_idx(g, s):
        pltpu.async_copy(idx_hbm.at[blk0 + g], iv.at[s], sem_i[s])

    def wait_idx(s):
        pltpu.make_async_copy(idx_hbm.at[0], iv.at[s], sem_i[s]).wait()

    def issue_gather(s):
        pltpu.async_copy(h_hbm.at[cid].at[iv.at[s, 0]], buf.at[s], sem_g[s])

    def wait_gather(s):
        pltpu.make_async_copy(h_hbm.at[cid].at[iv.at[s, 0]], buf.at[s],
                              sem_g[s]).wait()

    def issue_scatter(s):
        pltpu.async_copy(buf.at[s], num_acc.at[sdst_v.at[s]], sem_s[s],
                         add=True)

    def wait_scatter(s):
        pltpu.make_async_copy(buf.at[s], num_acc.at[sdst_v.at[s]],
                              sem_s[s]).wait()

    def compute_ex(s):
        def ex_step(j, _):
            s16 = iv[s, 0, pl.ds(j * 16, 16)]
            d16 = iv[s, 1, pl.ds(j * 16, 16)]
            e = plsc.load_gather(as_v, [s16]) + plsc.load_gather(ad_v, [d16])
            e = jnp.where(e > 0.0, e, 0.2 * e)
            ex_v[pl.ds(j * 16, 16)] = jnp.exp(e)
            sdst_v[s, pl.ds(j * 16, 16)] = d16
            return 0

        lax.fori_loop(0, C // 16, ex_step, 0)

    def scale(s):
        def sc_step(j, _):
            ex16 = ex_v[pl.ds(j * 16, 16)]
            for l in range(16):
                exs = ex16[l]
                eb = j * 16 + l
                for r in range(DP // 16):
                    buf[s, eb, pl.ds(r * 16, 16)] = (
                        buf[s, eb, pl.ds(r * 16, 16)] * exs)
            return 0

        lax.fori_loop(0, C // 16, sc_step, 0)

    issue_idx(0, 0)
    issue_idx(1, 1)
    wait_idx(0)
    issue_gather(0)

    def steady_chunk(g, p):
        pn = 1 - p
        wait_idx(pn)
        compute_ex(p)
        wait_gather(p)
        issue_idx(g + 2, p)
        wait_scatter(pn)
        issue_gather(pn)
        scale(p)
        issue_scatter(p)

    wait_idx(1)
    compute_ex(0)
    wait_gather(0)
    issue_idx(2, 0)
    issue_gather(1)
    scale(0)
    issue_scatter(0)
    wait_idx(0)
    compute_ex(1)
    wait_gather(1)
    issue_idx(3, 1)
    wait_scatter(0)
    issue_gather(0)
    scale(1)
    issue_scatter(1)

    def steady(i, _):
        for b in range(2):
            steady_chunk(2 + 2 * i + b, b)
        return 0

    lax.fori_loop(0, (N_CHUNKS - 3) // 2, steady, 0)

    compute_ex(0)
    wait_gather(0)
    wait_scatter(1)
    scale(0)
    issue_scatter(0)
    wait_scatter(0)

    plsc.subcore_barrier()
    pltpu.sync_copy(num_acc.at[pl.ds(row0, ROWS_PER_TILE)],
                    num_out.at[cid, pl.ds(row0, ROWS_PER_TILE)])


@functools.partial(
    pl.kernel,
    out_type=[
        jax.ShapeDtypeStruct((2, N_PAD, DP), jnp.float32),
    ],
    mesh=plsc.VectorSubcoreMesh(core_axis_name="c", subcore_axis_name="s",
                                num_cores=2, num_subcores=16),
    compiler_params=pltpu.CompilerParams(needs_layout_passes=False,
                                         use_tc_tiling_on_sc=False),
    scratch_types=[
        pltpu.VMEM((N_PAD,), jnp.float32),
        pltpu.VMEM((N_PAD,), jnp.float32),
        pltpu.VMEM((2, 2, C), jnp.int32),
        pltpu.VMEM((2, C), jnp.int32),
        pltpu.VMEM((C,), jnp.float32),
        pltpu.VMEM((2, C, DP), jnp.float32),
        pltpu.VMEM_SHARED((N_PAD, DP), jnp.float32),
        pltpu.SemaphoreType.DMA,
        pltpu.SemaphoreType.DMA,
        pltpu.SemaphoreType.DMA,
        pltpu.SemaphoreType.DMA,
        pltpu.SemaphoreType.DMA,
        pltpu.SemaphoreType.DMA,
    ],
)
def _edge_pass(h, alpha_s, alpha_d, idx,
               num_out,
               as_v, ad_v, iv, sdst_v, ex_v, buf,
               num_acc,
               sem_i0, sem_i1, sem_g0, sem_g1, sem_s0, sem_s1):
    _edge_body(h, alpha_s, alpha_d, idx, num_out,
               as_v, ad_v, iv, sdst_v, ex_v, buf, num_acc,
               sem_i0, sem_i1, sem_g0, sem_g1, sem_s0, sem_s1)


def kernel(x, edge_index, W1, a_s1, a_d1, b1, W2, a_s2, a_d2, b2):
    x_pad = jnp.zeros((N_PAD, D), jnp.float32).at[:N].set(x)
    src = edge_index[0].astype(jnp.int32)
    dst = edge_index[1].astype(jnp.int32)
    pad = jnp.full((N_BLKS * C - E,), N, jnp.int32)
    src = jnp.concatenate([src, pad]).reshape(N_BLKS, 1, C)
    dst = jnp.concatenate([dst, pad]).reshape(N_BLKS, 1, C)
    idx = jnp.concatenate([src, dst], axis=1)

    b1r = b1.reshape(1, D)
    b2r = b2.reshape(1, D)

    h1, pas1, pad1 = _proj(x_pad, W1, a_s1.reshape(D, 1), a_d1.reshape(D, 1))
    num1, = _edge_pass(h1, pas1.reshape(N_PAD), pad1.reshape(N_PAD), idx)
    h2, pas2, pad2 = _mid(num1, b1r, W2, a_s2.reshape(D, 1), a_d2.reshape(D, 1))
    num2, = _edge_pass(h2, pas2.reshape(N_PAD), pad2.reshape(N_PAD), idx)
    out = _final(num2, b2r)
    return out[:N]
